# Initial kernel scaffold; baseline (speedup 1.0000x reference)
import jax
import jax.numpy as jnp
from jax.experimental import pallas as pl


def kernel(x, Wq, Wk, Wv, Wo, bo, Wqi, Wki, Ww):
    raise NotImplementedError("write your pallas kernel here")


# speedup vs baseline: 4.9237x; 4.9237x over previous
import math

import jax
import jax.numpy as jnp
from jax.experimental import pallas as pl

_T = 2048
_DM = 1024
_H, _DH = 16, 64
_HI, _DI = 4, 64
_TOPK = 64
_BQ = 256
_NBLK = _T // _BQ

_NEG_INF = float("-inf")


def _bdot(a, b, dims=(((1,), (0,)), ((), ()))):
    return jax.lax.dot_general(a.astype(jnp.bfloat16), b.astype(jnp.bfloat16),
                               dims, preferred_element_type=jnp.float32)


def _proj_kernel(x_ref, wq_ref, wk_ref, wv_ref, wqi_ref, wki_ref, ww_ref,
                 q_ref, k_ref, v_ref, qi_ref, ki_ref, wsm_ref):
    x = x_ref[...]
    q_ref[...] = _bdot(x, wq_ref[...])
    k_ref[...] = _bdot(x, wk_ref[...])
    v_ref[...] = _bdot(x, wv_ref[...])
    qi_ref[...] = _bdot(x, wqi_ref[...])
    ki_ref[...] = _bdot(x, wki_ref[...])
    wl = _bdot(x, ww_ref[...])
    lane = jax.lax.broadcasted_iota(jnp.int32, wl.shape, 1)
    wl = jnp.where(lane < _HI, wl, _NEG_INF)
    m = jnp.max(wl, axis=-1, keepdims=True)
    e = jnp.exp(wl - m)
    wsm_ref[...] = e / jnp.sum(e, axis=-1, keepdims=True)


def _mid_int32(lo, hi):
    return (lo >> 1) + (hi >> 1) + (lo & hi & 1)


def _attn_kernel(q_ref, qi_ref, wsm_ref, k_ref, v_ref, ki_ref, wo_ref, bo_ref,
                 out_ref):
    i = pl.program_id(0)
    bq, t = _BQ, _T

    ki = ki_ref[:, :_DI]
    wsm = wsm_ref[...].astype(jnp.bfloat16).astype(jnp.float32)
    s_idx = jnp.zeros((bq, t), jnp.float32)
    for h in range(_HI):
        qi_h = qi_ref[:, h * _DI:(h + 1) * _DI]
        dots = _bdot(qi_h, ki, (((1,), (1,)), ((), ())))
        raw = jnp.maximum(dots * (1.0 / math.sqrt(_DI)), 0.0)
        raw = raw.astype(jnp.bfloat16).astype(jnp.float32)
        s_idx = s_idx + wsm[:, h][:, None] * raw

    col = jax.lax.broadcasted_iota(jnp.int32, (bq, t), 1)
    row = i * bq + jax.lax.broadcasted_iota(jnp.int32, (bq, t), 0)
    causal = col <= row
    s_m = jnp.where(causal, s_idx, -1.0)

    s_bits = jax.lax.bitcast_convert_type(s_m, jnp.int32)
    enc = jnp.where(s_bits < 0, s_bits ^ 0x7FFFFFFF, s_bits)

    sent_enc = jnp.int32(-1065353217)
    lo = jnp.full((bq, 1), sent_enc - 1, jnp.int32)
    hi = jnp.full((bq, 1), 0x7F800000, jnp.int32)
    for _ in range(32):
        mid = _mid_int32(lo, hi)
        cnt = jnp.sum((enc > mid).astype(jnp.int32), axis=-1, keepdims=True)
        gt = cnt >= _TOPK
        lo = jnp.where(gt, mid, lo)
        hi = jnp.where(gt, hi, mid)
    kth = hi

    cnt_gt = jnp.sum((enc > kth).astype(jnp.int32), axis=-1, keepdims=True)
    need = _TOPK - cnt_gt
    is_tie = (enc == kth).astype(jnp.int32)
    clo = jnp.zeros((bq, 1), jnp.int32)
    chi = jnp.full((bq, 1), t, jnp.int32)
    for _ in range(12):
        mid = _mid_int32(clo, chi)
        cnt = jnp.sum(is_tie * (col < mid).astype(jnp.int32),
                      axis=-1, keepdims=True)
        ge = cnt >= need
        chi = jnp.where(ge, mid, chi)
        clo = jnp.where(ge, clo, mid)
    cutoff = chi

    sel = ((enc > kth) | ((enc == kth) & (col < cutoff))) & causal

    scale = 1.0 / math.sqrt(_DH)
    ctx_parts = []
    for h in range(_H):
        q_h = q_ref[:, h * _DH:(h + 1) * _DH]
        k_h = k_ref[:, h * _DH:(h + 1) * _DH]
        logits = _bdot(q_h, k_h, (((1,), (1,)), ((), ())))
        logits = jnp.where(sel, logits * scale, _NEG_INF)
        m = jnp.max(logits, axis=-1, keepdims=True)
        p = jnp.exp(logits - m)
        p = p / jnp.sum(p, axis=-1, keepdims=True)
        ctx_parts.append(_bdot(p, v_ref[:, h * _DH:(h + 1) * _DH]))
    ctx = jnp.concatenate(ctx_parts, axis=-1)
    out_ref[...] = _bdot(ctx, wo_ref[...]) + bo_ref[...]


@jax.jit
def _run(x2d, Wq, Wk, Wv, Wo, bo, Wqi, Wki_p, Ww_p):
    f32 = jnp.float32
    row_blk = lambda w: pl.BlockSpec((_BQ, w), lambda i: (i, 0))
    full = lambda a, b: pl.BlockSpec((a, b), lambda i: (0, 0))

    q, k, v, qi, ki, wsm = pl.pallas_call(
        _proj_kernel,
        grid=(_NBLK,),
        in_specs=[row_blk(_DM), full(_DM, _DM), full(_DM, _DM), full(_DM, _DM),
                  full(_DM, _HI * _DI), full(_DM, 128), full(_DM, 128)],
        out_specs=[row_blk(_DM), row_blk(_DM), row_blk(_DM),
                   row_blk(_HI * _DI), row_blk(128), row_blk(128)],
        out_shape=[jax.ShapeDtypeStruct((_T, _DM), f32),
                   jax.ShapeDtypeStruct((_T, _DM), f32),
                   jax.ShapeDtypeStruct((_T, _DM), f32),
                   jax.ShapeDtypeStruct((_T, _HI * _DI), f32),
                   jax.ShapeDtypeStruct((_T, 128), f32),
                   jax.ShapeDtypeStruct((_T, 128), f32)],
    )(x2d, Wq, Wk, Wv, Wqi, Wki_p, Ww_p)

    out = pl.pallas_call(
        _attn_kernel,
        grid=(_NBLK,),
        in_specs=[row_blk(_DM), row_blk(_HI * _DI), row_blk(128),
                  full(_T, _DM), full(_T, _DM), full(_T, 128),
                  full(_DM, _DM), pl.BlockSpec((1, _DM), lambda i: (0, 0))],
        out_specs=row_blk(_DM),
        out_shape=jax.ShapeDtypeStruct((_T, _DM), f32),
    )(q, qi, wsm, k, v, ki, Wo, bo.reshape(1, _DM))
    return out


def kernel(x, Wq, Wk, Wv, Wo, bo, Wqi, Wki, Ww):
    b, t, _ = x.shape
    x2d = x.reshape(t, _DM)
    Wki_p = jnp.pad(Wki, ((0, 0), (0, 128 - Wki.shape[1])))
    Ww_p = jnp.pad(Ww, ((0, 0), (0, 128 - Ww.shape[1])))
    out = _run(x2d, Wq, Wk, Wv, Wo, bo, Wqi, Wki_p, Ww_p)
    return out.reshape(b, t, _DM)
